# Initial kernel scaffold; baseline (speedup 1.0000x reference)
import jax
import jax.numpy as jnp
from jax.experimental import pallas as pl


def kernel(target_wids, context_wids, i_embeddings, o_embeddings):
    raise NotImplementedError("write your pallas kernel here")


# speedup vs baseline: 3.1222x; 3.1222x over previous
import functools

import numpy as np
import jax
import jax.numpy as jnp
from jax import lax
from jax.experimental import pallas as pl
from jax.experimental.pallas import tpu as pltpu
from jax.experimental.pallas import tpu_sc as plsc

VOCAB = 100000
EMB = 128
N_NEG = 128
BATCH = 4096
CTX = 20

NC, NS = 2, 16
NW = NC * NS
BPW = BATCH // NW
NVREG = EMB // 16

def _neg_wids():
    wids = jax.random.randint(jax.random.key(1234), (BATCH, N_NEG), 0, VOCAB - 1)
    return wids.astype(jnp.int32)


def _sc_scores(tgt, ctx, neg, i_emb, o_emb):
    mesh = plsc.VectorSubcoreMesh(
        core_axis_name="c", subcore_axis_name="s", num_cores=NC, num_subcores=NS)

    @functools.partial(
        pl.kernel,
        out_type=(jax.ShapeDtypeStruct((BATCH, 16), jnp.float32),
                  jax.ShapeDtypeStruct((BATCH, N_NEG), jnp.float32)),
        mesh=mesh,
        compiler_params=pltpu.CompilerParams(needs_layout_passes=False),
        scratch_types=[
            pltpu.VMEM((BPW, CTX), jnp.int32),
            pltpu.VMEM((BPW,), jnp.int32),
            pltpu.VMEM((BPW, N_NEG), jnp.int32),
            pltpu.VMEM((BPW, EMB), jnp.float32),
            pltpu.VMEM((CTX, EMB), jnp.float32),
            pltpu.VMEM((N_NEG, EMB), jnp.float32),
            pltpu.VMEM((BPW, 16), jnp.float32),
            pltpu.VMEM((BPW, N_NEG), jnp.float32),
            pltpu.VMEM((16, 16), jnp.float32),
            pltpu.SemaphoreType.DMA,
        ],
    )
    def k(tgt_h, ctx_h, neg_h, iemb_h, oemb_h, pos_o, negs_o,
          ctx_idx, tgt_idx, neg_idx, tgt_rows, ctx_rows, neg_rows,
          pos_v, negs_v, part_buf, sem):
        wid = lax.axis_index("s") * NC + lax.axis_index("c")
        base = wid * BPW
        pltpu.sync_copy(ctx_h.at[pl.ds(base, BPW)], ctx_idx)
        pltpu.sync_copy(tgt_h.at[pl.ds(base, BPW)], tgt_idx)
        pltpu.sync_copy(neg_h.at[pl.ds(base, BPW)], neg_idx)
        pltpu.async_copy(oemb_h.at[tgt_idx], tgt_rows, sem).wait()

        def batch_body(b, carry):
            pltpu.async_copy(iemb_h.at[ctx_idx.at[b]], ctx_rows, sem).wait()
            avgs = []
            for j in range(NVREG):
                sl = pl.ds(j * 16, 16)
                acc = ctx_rows[0, sl]
                for c in range(1, CTX):
                    acc = acc + ctx_rows[c, sl]
                avgs.append(acc / jnp.float32(CTX))
            pp = avgs[0] * tgt_rows[b, pl.ds(0, 16)]
            for j in range(1, NVREG):
                pp = pp + avgs[j] * tgt_rows[b, pl.ds(j * 16, 16)]
            pos_v[b, :] = pp

            pltpu.async_copy(oemb_h.at[neg_idx.at[b]], neg_rows, sem).wait()

            lane = lax.iota(jnp.int32, 16)

            def neg_body(g, c2):
                for r in range(16):
                    n = g * 16 + r
                    q = avgs[0] * neg_rows[n, pl.ds(0, 16)]
                    for j in range(1, NVREG):
                        q = q + avgs[j] * neg_rows[n, pl.ds(j * 16, 16)]
                    part_buf[r, :] = q
                acc = plsc.load_gather(part_buf, [lane, jnp.zeros((16,), jnp.int32)])
                for j in range(1, 16):
                    acc = acc + plsc.load_gather(
                        part_buf, [lane, jnp.full((16,), j, jnp.int32)])
                negs_v[b, pl.ds(g * 16, 16)] = acc
                return c2

            lax.fori_loop(0, N_NEG // 16, neg_body, 0)
            return carry

        lax.fori_loop(0, BPW, batch_body, 0)
        pltpu.sync_copy(pos_v, pos_o.at[pl.ds(base, BPW)])
        pltpu.sync_copy(negs_v, negs_o.at[pl.ds(base, BPW)])

    return k(tgt, ctx, neg, i_emb, o_emb)


def _logsig(x):
    return jnp.minimum(x, 0.0) - jnp.log(1.0 + jnp.exp(-jnp.abs(x)))


def _tc_loss(pos_s, neg_s):
    def body(pos_ref, neg_ref, out_ref):
        p = jnp.sum(pos_ref[...], axis=1)
        lp = jnp.sum(_logsig(p))
        ln = jnp.sum(_logsig(-neg_ref[...]))
        out_ref[0, 0] = -(lp + ln)

    out = pl.pallas_call(
        body,
        out_shape=jax.ShapeDtypeStruct((1, 1), jnp.float32),
        in_specs=[pl.BlockSpec(memory_space=pltpu.VMEM),
                  pl.BlockSpec(memory_space=pltpu.VMEM)],
        out_specs=pl.BlockSpec(memory_space=pltpu.SMEM),
    )(pos_s, neg_s)
    return out[0, 0]


def kernel(target_wids, context_wids, i_embeddings, o_embeddings):
    tgt = target_wids.astype(jnp.int32)
    ctx = context_wids.astype(jnp.int32)
    neg = _neg_wids()
    pos_s, neg_s = _sc_scores(tgt, ctx, neg,
                              i_embeddings.astype(jnp.float32),
                              o_embeddings.astype(jnp.float32))
    return _tc_loss(pos_s, neg_s)


# speedup vs baseline: 5.1849x; 1.6607x over previous
import functools

import numpy as np
import jax
import jax.numpy as jnp
from jax import lax
from jax.experimental import pallas as pl
from jax.experimental.pallas import tpu as pltpu
from jax.experimental.pallas import tpu_sc as plsc

VOCAB = 100000
EMB = 128
N_NEG = 128
BATCH = 4096
CTX = 20

NC, NS = 2, 16
NW = NC * NS
BPW = BATCH // NW
NVREG = EMB // 16

def _neg_wids():
    wids = jax.random.randint(jax.random.key(1234), (BATCH, N_NEG), 0, VOCAB - 1)
    return wids.astype(jnp.int32)


def _sc_scores(tgt, ctx, neg, i_emb, o_emb):
    mesh = plsc.VectorSubcoreMesh(
        core_axis_name="c", subcore_axis_name="s", num_cores=NC, num_subcores=NS)

    @functools.partial(
        pl.kernel,
        out_type=(jax.ShapeDtypeStruct((BATCH, 16), jnp.float32),
                  jax.ShapeDtypeStruct((BATCH, N_NEG), jnp.float32)),
        mesh=mesh,
        compiler_params=pltpu.CompilerParams(needs_layout_passes=False),
        scratch_types=[
            pltpu.VMEM((BPW, CTX), jnp.int32),
            pltpu.VMEM((BPW,), jnp.int32),
            pltpu.VMEM((BPW, N_NEG), jnp.int32),
            pltpu.VMEM((BPW, EMB), jnp.float32),
            pltpu.VMEM((CTX, EMB), jnp.float32),
            pltpu.VMEM((CTX, EMB), jnp.float32),
            pltpu.VMEM((N_NEG, EMB), jnp.float32),
            pltpu.VMEM((N_NEG, EMB), jnp.float32),
            pltpu.VMEM((BPW, 16), jnp.float32),
            pltpu.VMEM((BPW, N_NEG), jnp.float32),
            pltpu.VMEM((16, 16), jnp.float32),
            pltpu.SemaphoreType.DMA,
            pltpu.SemaphoreType.DMA,
            pltpu.SemaphoreType.DMA,
            pltpu.SemaphoreType.DMA,
            pltpu.SemaphoreType.DMA,
        ],
    )
    def k(tgt_h, ctx_h, neg_h, iemb_h, oemb_h, pos_o, negs_o,
          ctx_idx, tgt_idx, neg_idx, tgt_rows, ctx_rows0, ctx_rows1,
          neg_rows0, neg_rows1, pos_v, negs_v, part_buf,
          sem, sem_c0, sem_c1, sem_n0, sem_n1):
        wid = lax.axis_index("s") * NC + lax.axis_index("c")
        base = wid * BPW
        pltpu.sync_copy(ctx_h.at[pl.ds(base, BPW)], ctx_idx)
        pltpu.sync_copy(tgt_h.at[pl.ds(base, BPW)], tgt_idx)
        pltpu.sync_copy(neg_h.at[pl.ds(base, BPW)], neg_idx)

        sems_c = (sem_c0, sem_c1)
        sems_n = (sem_n0, sem_n1)
        ctx_bufs = (ctx_rows0, ctx_rows1)
        neg_bufs = (neg_rows0, neg_rows1)
        lane = lax.iota(jnp.int32, 16)

        def start_ctx(b, buf):
            bb = jnp.minimum(b, BPW - 1)
            pltpu.async_copy(iemb_h.at[ctx_idx.at[bb]], ctx_bufs[buf], sems_c[buf])

        def start_neg(b, buf):
            bb = jnp.minimum(b, BPW - 1)
            pltpu.async_copy(oemb_h.at[neg_idx.at[bb]], neg_bufs[buf], sems_n[buf])

        def wait_ctx(buf):
            pltpu.make_async_copy(
                iemb_h.at[ctx_idx.at[0]], ctx_bufs[buf], sems_c[buf]).wait()

        def wait_neg(buf):
            pltpu.make_async_copy(
                oemb_h.at[neg_idx.at[0]], neg_bufs[buf], sems_n[buf]).wait()

        def process(b, buf):
            wait_ctx(buf)
            avgs = []
            for j in range(NVREG):
                sl = pl.ds(j * 16, 16)
                acc = ctx_bufs[buf][0, sl]
                for c in range(1, CTX):
                    acc = acc + ctx_bufs[buf][c, sl]
                avgs.append(acc / jnp.float32(CTX))
            start_ctx(b + 2, buf)
            pp = avgs[0] * tgt_rows[b, pl.ds(0, 16)]
            for j in range(1, NVREG):
                pp = pp + avgs[j] * tgt_rows[b, pl.ds(j * 16, 16)]
            pos_v[b, :] = pp

            wait_neg(buf)

            def neg_body(g, c2):
                for r in range(16):
                    n = g * 16 + r
                    q = avgs[0] * neg_bufs[buf][n, pl.ds(0, 16)]
                    for j in range(1, NVREG):
                        q = q + avgs[j] * neg_bufs[buf][n, pl.ds(j * 16, 16)]
                    part_buf[r, :] = q
                acc = plsc.load_gather(part_buf, [lane, jnp.zeros((16,), jnp.int32)])
                for j in range(1, 16):
                    acc = acc + plsc.load_gather(
                        part_buf, [lane, jnp.full((16,), j, jnp.int32)])
                negs_v[b, pl.ds(g * 16, 16)] = acc
                return c2

            lax.fori_loop(0, N_NEG // 16, neg_body, 0)
            start_neg(b + 2, buf)

        start_ctx(jnp.int32(0), 0)
        start_neg(jnp.int32(0), 0)
        start_ctx(jnp.int32(1), 1)
        start_neg(jnp.int32(1), 1)
        pltpu.async_copy(oemb_h.at[tgt_idx], tgt_rows, sem).wait()

        def pair_body(g, carry):
            b = g * 2
            process(b, 0)
            process(b + 1, 1)
            return carry

        lax.fori_loop(0, BPW // 2, pair_body, 0)
        wait_ctx(0)
        wait_neg(0)
        wait_ctx(1)
        wait_neg(1)
        pltpu.sync_copy(pos_v, pos_o.at[pl.ds(base, BPW)])
        pltpu.sync_copy(negs_v, negs_o.at[pl.ds(base, BPW)])

    return k(tgt, ctx, neg, i_emb, o_emb)


def _logsig(x):
    return jnp.minimum(x, 0.0) - jnp.log(1.0 + jnp.exp(-jnp.abs(x)))


def _tc_loss(pos_s, neg_s):
    def body(pos_ref, neg_ref, out_ref):
        p = jnp.sum(pos_ref[...], axis=1)
        lp = jnp.sum(_logsig(p))
        ln = jnp.sum(_logsig(-neg_ref[...]))
        out_ref[0, 0] = -(lp + ln)

    out = pl.pallas_call(
        body,
        out_shape=jax.ShapeDtypeStruct((1, 1), jnp.float32),
        in_specs=[pl.BlockSpec(memory_space=pltpu.VMEM),
                  pl.BlockSpec(memory_space=pltpu.VMEM)],
        out_specs=pl.BlockSpec(memory_space=pltpu.SMEM),
    )(pos_s, neg_s)
    return out[0, 0]


def kernel(target_wids, context_wids, i_embeddings, o_embeddings):
    tgt = target_wids.astype(jnp.int32)
    ctx = context_wids.astype(jnp.int32)
    neg = _neg_wids()
    pos_s, neg_s = _sc_scores(tgt, ctx, neg,
                              i_embeddings.astype(jnp.float32),
                              o_embeddings.astype(jnp.float32))
    return _tc_loss(pos_s, neg_s)


# speedup vs baseline: 5.3107x; 1.0243x over previous
import functools

import numpy as np
import jax
import jax.numpy as jnp
from jax import lax
from jax.experimental import pallas as pl
from jax.experimental.pallas import tpu as pltpu
from jax.experimental.pallas import tpu_sc as plsc

VOCAB = 100000
EMB = 128
N_NEG = 128
BATCH = 4096
CTX = 20

NC, NS = 2, 16
NW = NC * NS
BPW = BATCH // NW
NVREG = EMB // 16

def _neg_wids():
    wids = jax.random.randint(jax.random.key(1234), (BATCH, N_NEG), 0, VOCAB - 1)
    return wids.astype(jnp.int32)


def _sc_scores(tgt, ctx, neg, i_emb, o_emb):
    mesh = plsc.VectorSubcoreMesh(
        core_axis_name="c", subcore_axis_name="s", num_cores=NC, num_subcores=NS)

    @functools.partial(
        pl.kernel,
        out_type=(jax.ShapeDtypeStruct((BATCH, 16), jnp.float32),
                  jax.ShapeDtypeStruct((BATCH, N_NEG), jnp.float32)),
        mesh=mesh,
        compiler_params=pltpu.CompilerParams(needs_layout_passes=False),
        scratch_types=[
            pltpu.VMEM((BPW, CTX), jnp.int32),
            pltpu.VMEM((BPW,), jnp.int32),
            pltpu.VMEM((BPW, N_NEG), jnp.int32),
            pltpu.VMEM((BPW, EMB), jnp.float32),
            pltpu.VMEM((CTX, EMB), jnp.float32),
            pltpu.VMEM((CTX, EMB), jnp.float32),
            pltpu.VMEM((N_NEG, EMB), jnp.float32),
            pltpu.VMEM((N_NEG, EMB), jnp.float32),
            pltpu.VMEM((BPW, 16), jnp.float32),
            pltpu.VMEM((BPW, N_NEG), jnp.float32),
            pltpu.VMEM((16, 17), jnp.float32),
            pltpu.SemaphoreType.DMA,
            pltpu.SemaphoreType.DMA,
            pltpu.SemaphoreType.DMA,
            pltpu.SemaphoreType.DMA,
            pltpu.SemaphoreType.DMA,
        ],
    )
    def k(tgt_h, ctx_h, neg_h, iemb_h, oemb_h, pos_o, negs_o,
          ctx_idx, tgt_idx, neg_idx, tgt_rows, ctx_rows0, ctx_rows1,
          neg_rows0, neg_rows1, pos_v, negs_v, part_buf,
          sem, sem_c0, sem_c1, sem_n0, sem_n1):
        wid = lax.axis_index("s") * NC + lax.axis_index("c")
        base = wid * BPW
        pltpu.sync_copy(ctx_h.at[pl.ds(base, BPW)], ctx_idx)
        pltpu.sync_copy(tgt_h.at[pl.ds(base, BPW)], tgt_idx)
        pltpu.sync_copy(neg_h.at[pl.ds(base, BPW)], neg_idx)

        sems_c = (sem_c0, sem_c1)
        sems_n = (sem_n0, sem_n1)
        ctx_bufs = (ctx_rows0, ctx_rows1)
        neg_bufs = (neg_rows0, neg_rows1)
        lane = lax.iota(jnp.int32, 16)

        def start_ctx(b, buf):
            bb = jnp.minimum(b, BPW - 1)
            pltpu.async_copy(iemb_h.at[ctx_idx.at[bb]], ctx_bufs[buf], sems_c[buf])

        def start_neg(b, buf):
            bb = jnp.minimum(b, BPW - 1)
            pltpu.async_copy(oemb_h.at[neg_idx.at[bb]], neg_bufs[buf], sems_n[buf])

        def wait_ctx(buf):
            pltpu.make_async_copy(
                iemb_h.at[ctx_idx.at[0]], ctx_bufs[buf], sems_c[buf]).wait()

        def wait_neg(buf):
            pltpu.make_async_copy(
                oemb_h.at[neg_idx.at[0]], neg_bufs[buf], sems_n[buf]).wait()

        def process(b, buf):
            wait_ctx(buf)
            avgs = []
            for j in range(NVREG):
                sl = pl.ds(j * 16, 16)
                terms = [ctx_bufs[buf][c, sl] for c in range(CTX)]
                while len(terms) > 1:
                    terms = ([terms[i] + terms[i + 1]
                              for i in range(0, len(terms) - 1, 2)]
                             + ([terms[-1]] if len(terms) % 2 else []))
                avgs.append(terms[0] / jnp.float32(CTX))
            start_ctx(b + 2, buf)
            pp = avgs[0] * tgt_rows[b, pl.ds(0, 16)]
            for j in range(1, NVREG):
                pp = pp + avgs[j] * tgt_rows[b, pl.ds(j * 16, 16)]
            pos_v[b, :] = pp

            wait_neg(buf)

            def neg_body(g, c2):
                for r in range(16):
                    n = g * 16 + r
                    prods = [avgs[j] * neg_bufs[buf][n, pl.ds(j * 16, 16)]
                             for j in range(NVREG)]
                    while len(prods) > 1:
                        prods = [prods[i] + prods[i + 1]
                                 for i in range(0, len(prods), 2)]
                    part_buf[r, pl.ds(0, 16)] = prods[0]
                acc = plsc.load_gather(part_buf, [lane, jnp.zeros((16,), jnp.int32)])
                for j in range(1, 16):
                    acc = acc + plsc.load_gather(
                        part_buf, [lane, jnp.full((16,), j, jnp.int32)])
                negs_v[b, pl.ds(g * 16, 16)] = acc
                return c2

            lax.fori_loop(0, N_NEG // 16, neg_body, 0)
            start_neg(b + 2, buf)

        start_ctx(jnp.int32(0), 0)
        start_neg(jnp.int32(0), 0)
        start_ctx(jnp.int32(1), 1)
        start_neg(jnp.int32(1), 1)
        pltpu.async_copy(oemb_h.at[tgt_idx], tgt_rows, sem).wait()

        def pair_body(g, carry):
            b = g * 2
            process(b, 0)
            process(b + 1, 1)
            return carry

        lax.fori_loop(0, BPW // 2, pair_body, 0)
        wait_ctx(0)
        wait_neg(0)
        wait_ctx(1)
        wait_neg(1)
        pltpu.sync_copy(pos_v, pos_o.at[pl.ds(base, BPW)])
        pltpu.sync_copy(negs_v, negs_o.at[pl.ds(base, BPW)])

    return k(tgt, ctx, neg, i_emb, o_emb)


def _logsig(x):
    return jnp.minimum(x, 0.0) - jnp.log(1.0 + jnp.exp(-jnp.abs(x)))


def _tc_loss(pos_s, neg_s):
    def body(pos_ref, neg_ref, out_ref):
        p = jnp.sum(pos_ref[...], axis=1)
        lp = jnp.sum(_logsig(p))
        ln = jnp.sum(_logsig(-neg_ref[...]))
        out_ref[0, 0] = -(lp + ln)

    out = pl.pallas_call(
        body,
        out_shape=jax.ShapeDtypeStruct((1, 1), jnp.float32),
        in_specs=[pl.BlockSpec(memory_space=pltpu.VMEM),
                  pl.BlockSpec(memory_space=pltpu.VMEM)],
        out_specs=pl.BlockSpec(memory_space=pltpu.SMEM),
    )(pos_s, neg_s)
    return out[0, 0]


def kernel(target_wids, context_wids, i_embeddings, o_embeddings):
    tgt = target_wids.astype(jnp.int32)
    ctx = context_wids.astype(jnp.int32)
    neg = _neg_wids()
    pos_s, neg_s = _sc_scores(tgt, ctx, neg,
                              i_embeddings.astype(jnp.float32),
                              o_embeddings.astype(jnp.float32))
    return _tc_loss(pos_s, neg_s)


# speedup vs baseline: 5.6086x; 1.0561x over previous
import functools

import numpy as np
import jax
import jax.numpy as jnp
from jax import lax
from jax.experimental import pallas as pl
from jax.experimental.pallas import tpu as pltpu
from jax.experimental.pallas import tpu_sc as plsc

VOCAB = 100000
EMB = 128
N_NEG = 128
BATCH = 4096
CTX = 20

NC, NS = 2, 16
NW = NC * NS
BPW = BATCH // NW
NVREG = EMB // 16

def _neg_wids():
    wids = jax.random.randint(jax.random.key(1234), (BATCH, N_NEG), 0, VOCAB - 1)
    return wids.astype(jnp.int32)


def _sc_scores(tgt, ctx, neg, i_emb, o_emb):
    mesh = plsc.VectorSubcoreMesh(
        core_axis_name="c", subcore_axis_name="s", num_cores=NC, num_subcores=NS)

    @functools.partial(
        pl.kernel,
        out_type=(jax.ShapeDtypeStruct((BATCH, 16), jnp.float32),
                  jax.ShapeDtypeStruct((BATCH, N_NEG), jnp.float32)),
        mesh=mesh,
        compiler_params=pltpu.CompilerParams(needs_layout_passes=False),
        scratch_types=[
            pltpu.VMEM((BPW, CTX), jnp.int32),
            pltpu.VMEM((BPW,), jnp.int32),
            pltpu.VMEM((BPW, N_NEG), jnp.int32),
            pltpu.VMEM((BPW, EMB), jnp.float32),
            pltpu.VMEM((CTX, EMB), jnp.float32),
            pltpu.VMEM((CTX, EMB), jnp.float32),
            pltpu.VMEM((N_NEG, EMB), jnp.float32),
            pltpu.VMEM((N_NEG, EMB), jnp.float32),
            pltpu.VMEM((BPW, 16), jnp.float32),
            pltpu.VMEM((N_NEG,), jnp.float32),
            pltpu.VMEM((N_NEG,), jnp.float32),
            pltpu.VMEM((N_NEG, 17), jnp.float32),
            pltpu.SemaphoreType.DMA,
            pltpu.SemaphoreType.DMA,
            pltpu.SemaphoreType.DMA,
            pltpu.SemaphoreType.DMA,
            pltpu.SemaphoreType.DMA,
            pltpu.SemaphoreType.DMA,
            pltpu.SemaphoreType.DMA,
        ],
    )
    def k(tgt_h, ctx_h, neg_h, iemb_h, oemb_h, pos_o, negs_o,
          ctx_idx, tgt_idx, neg_idx, tgt_rows, ctx_rows0, ctx_rows1,
          neg_rows0, neg_rows1, pos_v, negs_v0, negs_v1, part_buf,
          sem, sem_c0, sem_c1, sem_n0, sem_n1, sem_o0, sem_o1):
        wid = lax.axis_index("s") * NC + lax.axis_index("c")
        base = wid * BPW
        pltpu.sync_copy(ctx_h.at[pl.ds(base, BPW)], ctx_idx)
        pltpu.sync_copy(tgt_h.at[pl.ds(base, BPW)], tgt_idx)
        pltpu.sync_copy(neg_h.at[pl.ds(base, BPW)], neg_idx)

        sems_c = (sem_c0, sem_c1)
        sems_n = (sem_n0, sem_n1)
        sems_o = (sem_o0, sem_o1)
        ctx_bufs = (ctx_rows0, ctx_rows1)
        neg_bufs = (neg_rows0, neg_rows1)
        out_bufs = (negs_v0, negs_v1)
        lane = lax.iota(jnp.int32, 16)

        def start_ctx(b, buf):
            bb = jnp.minimum(b, BPW - 1)
            pltpu.async_copy(iemb_h.at[ctx_idx.at[bb]], ctx_bufs[buf], sems_c[buf])

        def start_neg(b, buf):
            bb = jnp.minimum(b, BPW - 1)
            pltpu.async_copy(oemb_h.at[neg_idx.at[bb]], neg_bufs[buf], sems_n[buf])

        def wait_ctx(buf):
            pltpu.make_async_copy(
                iemb_h.at[ctx_idx.at[0]], ctx_bufs[buf], sems_c[buf]).wait()

        def wait_neg(buf):
            pltpu.make_async_copy(
                oemb_h.at[neg_idx.at[0]], neg_bufs[buf], sems_n[buf]).wait()

        def process(b, buf):
            wait_ctx(buf)
            avgs = []
            for j in range(NVREG):
                sl = pl.ds(j * 16, 16)
                terms = [ctx_bufs[buf][c, sl] for c in range(CTX)]
                while len(terms) > 1:
                    terms = ([terms[i] + terms[i + 1]
                              for i in range(0, len(terms) - 1, 2)]
                             + ([terms[-1]] if len(terms) % 2 else []))
                avgs.append(terms[0] / jnp.float32(CTX))
            start_ctx(b + 2, buf)
            pp = avgs[0] * tgt_rows[b, pl.ds(0, 16)]
            for j in range(1, NVREG):
                pp = pp + avgs[j] * tgt_rows[b, pl.ds(j * 16, 16)]
            pos_v[b, :] = pp

            wait_neg(buf)
            @pl.when(b >= 2)
            def _():
                pltpu.make_async_copy(
                    out_bufs[buf], negs_o.at[base], sems_o[buf]).wait()

            def neg_group(g):
                base_r = g * 16
                for r in range(16):
                    n = base_r + r
                    prods = [avgs[j] * neg_bufs[buf][n, pl.ds(j * 16, 16)]
                             for j in range(NVREG)]
                    while len(prods) > 1:
                        prods = [prods[i] + prods[i + 1]
                                 for i in range(0, len(prods), 2)]
                    part_buf[n, pl.ds(0, 16)] = prods[0]
                row_idx = base_r + lane
                cols = [plsc.load_gather(
                            part_buf, [row_idx, jnp.full((16,), j, jnp.int32)])
                        for j in range(16)]
                while len(cols) > 1:
                    cols = [cols[i] + cols[i + 1] for i in range(0, len(cols), 2)]
                out_bufs[buf][pl.ds(base_r, 16)] = cols[0]

            plsc.parallel_loop(0, N_NEG // 16, 1)(neg_group)
            pltpu.async_copy(out_bufs[buf], negs_o.at[base + b], sems_o[buf])
            start_neg(b + 2, buf)

        start_ctx(jnp.int32(0), 0)
        start_neg(jnp.int32(0), 0)
        start_ctx(jnp.int32(1), 1)
        start_neg(jnp.int32(1), 1)
        pltpu.async_copy(oemb_h.at[tgt_idx], tgt_rows, sem).wait()

        def pair_body(g, carry):
            b = g * 2
            process(b, 0)
            process(b + 1, 1)
            return carry

        lax.fori_loop(0, BPW // 2, pair_body, 0)
        wait_ctx(0)
        wait_neg(0)
        wait_ctx(1)
        wait_neg(1)
        pltpu.make_async_copy(negs_v0, negs_o.at[base], sem_o0).wait()
        pltpu.make_async_copy(negs_v1, negs_o.at[base], sem_o1).wait()
        pltpu.sync_copy(pos_v, pos_o.at[pl.ds(base, BPW)])

    return k(tgt, ctx, neg, i_emb, o_emb)


def _logsig(x):
    return jnp.minimum(x, 0.0) - jnp.log(1.0 + jnp.exp(-jnp.abs(x)))


def _tc_loss(pos_s, neg_s):
    def body(pos_ref, neg_ref, out_ref):
        p = jnp.sum(pos_ref[...], axis=1)
        lp = jnp.sum(_logsig(p))
        ln = jnp.sum(_logsig(-neg_ref[...]))
        out_ref[0, 0] = -(lp + ln)

    out = pl.pallas_call(
        body,
        out_shape=jax.ShapeDtypeStruct((1, 1), jnp.float32),
        in_specs=[pl.BlockSpec(memory_space=pltpu.VMEM),
                  pl.BlockSpec(memory_space=pltpu.VMEM)],
        out_specs=pl.BlockSpec(memory_space=pltpu.SMEM),
    )(pos_s, neg_s)
    return out[0, 0]


def kernel(target_wids, context_wids, i_embeddings, o_embeddings):
    tgt = target_wids.astype(jnp.int32)
    ctx = context_wids.astype(jnp.int32)
    neg = _neg_wids()
    pos_s, neg_s = _sc_scores(tgt, ctx, neg,
                              i_embeddings.astype(jnp.float32),
                              o_embeddings.astype(jnp.float32))
    return _tc_loss(pos_s, neg_s)
